# baseline (speedup 1.0000x reference)
import functools

import jax
import jax.numpy as jnp
from jax import lax
from jax.experimental import pallas as pl
from jax.experimental.pallas import tpu as pltpu
from jax.experimental.pallas import tpu_sc as plsc

BN_EPS = 1e-3
LN_EPS = 1e-3

_NC = 2
_NS = 16
_NW = _NC * _NS
_LANES = 16


def _sc_pool_kernel(B, L, V, D, CB, NCH):
    G = CB * L
    RPW = B // _NW

    mesh = plsc.VectorSubcoreMesh(core_axis_name="c", subcore_axis_name="s")

    @functools.partial(
        pl.kernel,
        out_type=jax.ShapeDtypeStruct((B, D), jnp.float32),
        mesh=mesh,
        scratch_types=[
            pltpu.VMEM((NCH, G), jnp.int32),
            pltpu.VMEM((G, D), jnp.float32),
            pltpu.VMEM((RPW, D), jnp.float32),
            pltpu.SemaphoreType.DMA,
        ],
        compiler_params=pltpu.CompilerParams(use_tc_tiling_on_sc=False),
    )
    def sc_pool(idx_hbm, table_hbm, out_hbm, idx_v, rows_v, acc_v, sem):
        wid = lax.axis_index("s") * _NC + lax.axis_index("c")
        pltpu.sync_copy(idx_hbm.at[pl.ds(wid * NCH, NCH)], idx_v)

        nvec = D // _LANES

        def chunk_body(j, carry):
            pltpu.async_copy(table_hbm.at[idx_v.at[j]], rows_v, sem).wait()

            def red_body(l, accs):
                new = []
                for bb in range(CB):
                    for d in range(nvec):
                        new.append(
                            accs[bb * nvec + d]
                            + rows_v[bb * L + l, pl.ds(d * _LANES, _LANES)]
                        )
                return tuple(new)

            zero = jnp.zeros((_LANES,), jnp.float32)
            accs = lax.fori_loop(0, L, red_body, (zero,) * (CB * nvec))
            for bb in range(CB):
                for d in range(nvec):
                    acc_v[j * CB + bb, pl.ds(d * _LANES, _LANES)] = accs[bb * nvec + d]
            return carry

        lax.fori_loop(0, NCH, chunk_body, 0)
        pltpu.sync_copy(acc_v, out_hbm.at[pl.ds(wid * RPW, RPW)])

    return sc_pool


def _tc_dense_body(x_ref, w_ref, b_ref, bng_ref, bnb_ref, bnm_ref, bnv_ref,
                   lng_ref, lnb_ref, inv_l_ref, o_ref):
    x = x_ref[...] * inv_l_ref[0, 0]
    y = jnp.dot(x, w_ref[...], preferred_element_type=jnp.float32,
                precision=jax.lax.Precision.HIGHEST)
    y = y + b_ref[...]
    y = bng_ref[...] * (y - bnm_ref[...]) * jax.lax.rsqrt(bnv_ref[...] + BN_EPS)
    y = y + bnb_ref[...]
    mu = jnp.mean(y, axis=-1, keepdims=True)
    var = jnp.mean((y - mu) ** 2, axis=-1, keepdims=True)
    o_ref[...] = lng_ref[...] * (y - mu) * jax.lax.rsqrt(var + LN_EPS) + lnb_ref[...]


def kernel(inputs, table, W, b, bn_gamma, bn_beta, bn_mean, bn_var, ln_gamma, ln_beta):
    B, L = inputs.shape
    V, D = table.shape

    CB = max(1, 128 // L)
    while (B // _NW) % CB:
        CB -= 1
    NCH = B // (_NW * CB)

    idx = inputs.astype(jnp.int32).reshape(_NW * NCH, CB * L)
    sums = _sc_pool_kernel(B, L, V, D, CB, NCH)(idx, table)

    row = lambda v: v.reshape(1, D).astype(jnp.float32)
    inv_l = jnp.full((1, 1), 1.0 / L, dtype=jnp.float32)
    out = pl.pallas_call(
        _tc_dense_body,
        out_shape=jax.ShapeDtypeStruct((B, D), jnp.float32),
    )(sums, W, row(b), row(bn_gamma), row(bn_beta), row(bn_mean), row(bn_var),
      row(ln_gamma), row(ln_beta), inv_l)
    return out
